# Initial kernel scaffold; baseline (speedup 1.0000x reference)
import jax
import jax.numpy as jnp
from jax.experimental import pallas as pl


def kernel(x, edge_index, W1, b1, W2, b2):
    raise NotImplementedError("write your pallas kernel here")


# speedup vs baseline: 4.6872x; 4.6872x over previous
import functools

import jax
import jax.numpy as jnp
from jax import lax
from jax.experimental import pallas as pl
from jax.experimental.pallas import tpu as pltpu
from jax.experimental.pallas import tpu_sc as plsc

N = 10000
E = 320000
F = 128
C = 64

NC = 2
NS = 16
NW = NC * NS

CD = 64
ND = 158
EPT = ND * CD
EPAD = EPT * NW
CP = 128
NT = EPAD // NS // CP

NP = 10240
RPT = NP // NS

HALF = N // NC
NPH = 5632
RPH = NPH // NS
TRASHD = N
TRASHP = HALF

_mesh = plsc.VectorSubcoreMesh(
    core_axis_name="c", subcore_axis_name="s", num_cores=NC, num_subcores=NS
)


def _deg_body(edge_ref, degs_ref, src2, dst2, w2, zb, acc):
    c = lax.axis_index("c")
    s = lax.axis_index("s")
    wid = s * NC + c
    pltpu.sync_copy(edge_ref.at[0, wid], src2)
    pltpu.sync_copy(edge_ref.at[1, wid], dst2)

    def prep(j, carry):
        for q in range(CD // 16):
            sv = src2[j, pl.ds(q * 16, 16)]
            dv = dst2[j, pl.ds(q * 16, 16)]
            w2[j, pl.ds(q * 16, 16)] = jnp.where(sv == dv, 0.0, 1.0).astype(
                jnp.float32
            )
        return carry

    lax.fori_loop(0, ND, prep, 0)

    for q in range(CD // 16):
        zb[pl.ds(q * 16, 16)] = jnp.zeros((16,), jnp.float32)
    base_r = s * RPT
    for k in range(RPT // CD):
        pltpu.sync_copy(zb, acc.at[pl.ds(base_r + k * CD, CD)])
    plsc.subcore_barrier()

    def scat(j, carry):
        pltpu.sync_copy(w2.at[j], acc.at[src2.at[j]], add=True)
        return carry

    lax.fori_loop(0, ND, scat, 0)
    plsc.subcore_barrier()
    pltpu.sync_copy(acc.at[pl.ds(base_r, RPT)], degs_ref.at[c, pl.ds(base_r, RPT)])


_deg_call = functools.partial(
    pl.kernel,
    out_type=jax.ShapeDtypeStruct((NC, NP), jnp.float32),
    mesh=_mesh,
    scratch_types=[
        pltpu.VMEM((ND, CD), jnp.int32),
        pltpu.VMEM((ND, CD), jnp.int32),
        pltpu.VMEM((ND, CD), jnp.float32),
        pltpu.VMEM((CD,), jnp.float32),
        pltpu.VMEM_SHARED((NP,), jnp.float32),
    ],
)(_deg_body)


def _prop_body(g_ref, edge_ref, parts_ref, srcb, dstb, dstm, rows0, rows1, acc,
               semi, semg0, semg1):
    c = lax.axis_index("c")
    s = lax.axis_index("s")
    lo = c * HALF

    def idx_start(j, slot):
        pltpu.async_copy(edge_ref.at[0, s, j], srcb.at[slot], semi)
        pltpu.async_copy(edge_ref.at[1, s, j], dstb.at[slot], semi)

    def idx_wait(slot):
        pltpu.make_async_copy(edge_ref.at[0, 0, 0], srcb.at[slot], semi).wait()
        pltpu.make_async_copy(edge_ref.at[0, 0, 0], dstb.at[slot], semi).wait()

    def remap(slot):
        for q in range(CP // 16):
            sv = srcb[slot, pl.ds(q * 16, 16)]
            dv = dstb[slot, pl.ds(q * 16, 16)]
            loc = dv - lo
            bad = jnp.logical_or(
                jnp.logical_or(loc < 0, loc >= HALF), sv == dv
            )
            dstm[slot, pl.ds(q * 16, 16)] = jnp.where(bad, TRASHP, loc)

    def gstart(slot, buf, sem):
        pltpu.async_copy(g_ref.at[srcb.at[slot]], buf, sem)

    def gwait(buf, sem):
        pltpu.make_async_copy(g_ref.at[pl.ds(0, CP)], buf, sem).wait()

    def zrow(r, carry):
        for q in range(F // 16):
            rows0[r, pl.ds(q * 16, 16)] = jnp.zeros((16,), jnp.float32)
        return carry

    lax.fori_loop(0, CP, zrow, 0)
    base_r = s * RPH
    pltpu.sync_copy(rows0, acc.at[pl.ds(base_r, CP)])
    pltpu.sync_copy(rows0, acc.at[pl.ds(base_r + CP, CP)])
    pltpu.sync_copy(rows0.at[pl.ds(0, RPH - 2 * CP)],
                    acc.at[pl.ds(base_r + 2 * CP, RPH - 2 * CP)])
    plsc.subcore_barrier()

    idx_start(0, 0)
    idx_wait(0)
    remap(0)
    gstart(0, rows0, semg0)
    idx_start(1, 1)

    def body(j, carry):
        s0 = lax.rem(j, 2)
        s1 = lax.rem(j + 1, 2)
        even = s0 == 0

        @pl.when(j + 1 < NT)
        def _():
            idx_wait(s1)
            remap(s1)

            @pl.when(even)
            def _():
                gstart(s1, rows1, semg1)

            @pl.when(jnp.logical_not(even))
            def _():
                gstart(s1, rows0, semg0)

        @pl.when(even)
        def _():
            gwait(rows0, semg0)

        @pl.when(jnp.logical_not(even))
        def _():
            gwait(rows1, semg1)

        @pl.when(j + 2 < NT)
        def _():
            idx_start(j + 2, s0)

        @pl.when(even)
        def _():
            pltpu.sync_copy(rows0, acc.at[dstm.at[s0]], add=True)

        @pl.when(jnp.logical_not(even))
        def _():
            pltpu.sync_copy(rows1, acc.at[dstm.at[s0]], add=True)

        return carry

    lax.fori_loop(0, NT, body, 0)
    plsc.subcore_barrier()
    pltpu.sync_copy(acc.at[pl.ds(base_r, RPH)], parts_ref.at[c, pl.ds(base_r, RPH)])


_prop = functools.partial(
    pl.kernel,
    out_type=jax.ShapeDtypeStruct((NC, NPH, F), jnp.float32),
    mesh=_mesh,
    scratch_types=[
        pltpu.VMEM((2, CP), jnp.int32),
        pltpu.VMEM((2, CP), jnp.int32),
        pltpu.VMEM((2, CP), jnp.int32),
        pltpu.VMEM((CP, F), jnp.float32),
        pltpu.VMEM((CP, F), jnp.float32),
        pltpu.VMEM_SHARED((NPH, F), jnp.float32),
        pltpu.SemaphoreType.DMA,
        pltpu.SemaphoreType.DMA,
        pltpu.SemaphoreType.DMA,
    ],
)(_prop_body)


_BR = 200
_GRID = N // _BR
_BPH = HALF // _BR

_ROWS_F = pl.BlockSpec((_BR, F), lambda i: (i, 0))
_PARTS = pl.BlockSpec((1, _BR, F), lambda i: (i // _BPH, i % _BPH, 0))
_F_SHAPE = jax.ShapeDtypeStruct((N, F), jnp.float32)


def _tca_body(degp_ref, x_ref, d_ref, g_ref):
    deg = degp_ref[0] + degp_ref[1]
    dis = jnp.where(deg > 0.0, lax.rsqrt(jnp.maximum(deg, 1e-12)), 0.0)
    dmat = jnp.broadcast_to(dis, (_BR, F))
    d_ref[...] = dmat
    g_ref[...] = dmat * x_ref[...]


def _tca(degp, x):
    return pl.pallas_call(
        _tca_body,
        grid=(_GRID,),
        in_specs=[
            pl.BlockSpec((NC, _BR, 1), lambda i: (0, i, 0)),
            _ROWS_F,
        ],
        out_specs=[_ROWS_F, _ROWS_F],
        out_shape=[_F_SHAPE, _F_SHAPE],
    )(degp, x)


def _tcb_body(p_ref, d_ref, tx_ref, g_ref):
    tx = -(d_ref[...] * p_ref[0])
    tx_ref[...] = tx
    g_ref[...] = d_ref[...] * tx


def _tcb(parts, d):
    return pl.pallas_call(
        _tcb_body,
        grid=(_GRID,),
        in_specs=[_PARTS, _ROWS_F],
        out_specs=[_ROWS_F, _ROWS_F],
        out_shape=[_F_SHAPE, _F_SHAPE],
    )(parts, d)


def _tcc1_body(x_ref, t1_ref, p_ref, d_ref, w_ref, b_ref, h_ref, g_ref):
    r = -2.0 * d_ref[...] * p_ref[0]
    acc = jnp.dot(x_ref[...], w_ref[0] - w_ref[2],
                  preferred_element_type=jnp.float32)
    acc += jnp.dot(t1_ref[...], w_ref[1], preferred_element_type=jnp.float32)
    acc += jnp.dot(r, w_ref[2], preferred_element_type=jnp.float32)
    acc += b_ref[...]
    h = jnp.maximum(acc, 0.0)
    h_ref[...] = h
    g_ref[...] = d_ref[...] * h


def _tcc1(x, t1, parts, d, w, b):
    return pl.pallas_call(
        _tcc1_body,
        grid=(_GRID,),
        in_specs=[
            _ROWS_F,
            _ROWS_F,
            _PARTS,
            _ROWS_F,
            pl.BlockSpec((3, F, F), lambda i: (0, 0, 0)),
            pl.BlockSpec((1, F), lambda i: (0, 0)),
        ],
        out_specs=[_ROWS_F, _ROWS_F],
        out_shape=[_F_SHAPE, _F_SHAPE],
    )(x, t1, parts, d, w, b)


def _tcc2_body(h_ref, t1_ref, p_ref, d_ref, w_ref, b_ref, o_ref):
    r = -2.0 * d_ref[...] * p_ref[0]
    acc = jnp.dot(h_ref[...], w_ref[0] - w_ref[2],
                  preferred_element_type=jnp.float32)
    acc += jnp.dot(t1_ref[...], w_ref[1], preferred_element_type=jnp.float32)
    acc += jnp.dot(r, w_ref[2], preferred_element_type=jnp.float32)
    acc += b_ref[...]
    m = jnp.max(acc, axis=1, keepdims=True)
    ex = jnp.exp(acc - m)
    lse = jnp.log(jnp.sum(ex, axis=1, keepdims=True)) + m
    o_ref[...] = acc - lse


def _tcc2(h, t1, parts, d, w, b):
    return pl.pallas_call(
        _tcc2_body,
        grid=(_GRID,),
        in_specs=[
            _ROWS_F,
            _ROWS_F,
            _PARTS,
            _ROWS_F,
            pl.BlockSpec((3, F, C), lambda i: (0, 0, 0)),
            pl.BlockSpec((1, C), lambda i: (0, 0)),
        ],
        out_specs=pl.BlockSpec((_BR, C), lambda i: (i, 0)),
        out_shape=jax.ShapeDtypeStruct((N, C), jnp.float32),
    )(h, t1, parts, d, w, b)


def kernel(x, edge_index, W1, b1, W2, b2):
    ei = jnp.pad(edge_index, ((0, 0), (0, EPAD - E)))
    ei_deg = ei.reshape(2, NW, ND, CD)
    ei_prop = ei.reshape(2, NS, NT, CP)
    degp = _deg_call(ei_deg)
    d, g0 = _tca(degp.reshape(NC, NP, 1), x)
    p = _prop(g0, ei_prop)
    tx1, g1 = _tcb(p, d)
    q = _prop(g1, ei_prop)
    h, gh = _tcc1(x, tx1, q, d, W1, b1.reshape(1, F))
    r1 = _prop(gh, ei_prop)
    ty1, g3 = _tcb(r1, d)
    z = _prop(g3, ei_prop)
    return _tcc2(h, ty1, z, d, W2, b2.reshape(1, C))


# speedup vs baseline: 5.1448x; 1.0976x over previous
import functools

import jax
import jax.numpy as jnp
from jax import lax
from jax.experimental import pallas as pl
from jax.experimental.pallas import tpu as pltpu
from jax.experimental.pallas import tpu_sc as plsc

N = 10000
E = 320000
F = 128
C = 64

NC = 2
NS = 16
NW = NC * NS

CD = 64
ND = 158
EPT = ND * CD
EPAD = EPT * NW
CP = 128
NT = EPAD // NS // CP

NP = 10240
RPT = NP // NS

HALF = N // NC
NPH = 5632
RPH = NPH // NS
TRASHD = N
TRASHP = HALF

_mesh = plsc.VectorSubcoreMesh(
    core_axis_name="c", subcore_axis_name="s", num_cores=NC, num_subcores=NS
)


def _deg_body(edge_ref, degs_ref, src2, dst2, w2, zb, acc):
    c = lax.axis_index("c")
    s = lax.axis_index("s")
    wid = s * NC + c
    pltpu.sync_copy(edge_ref.at[0, wid], src2)
    pltpu.sync_copy(edge_ref.at[1, wid], dst2)

    def prep(j, carry):
        for q in range(CD // 16):
            sv = src2[j, pl.ds(q * 16, 16)]
            dv = dst2[j, pl.ds(q * 16, 16)]
            w2[j, pl.ds(q * 16, 16)] = jnp.where(sv == dv, 0.0, 1.0).astype(
                jnp.float32
            )
        return carry

    lax.fori_loop(0, ND, prep, 0)

    for q in range(CD // 16):
        zb[pl.ds(q * 16, 16)] = jnp.zeros((16,), jnp.float32)
    base_r = s * RPT
    for k in range(RPT // CD):
        pltpu.sync_copy(zb, acc.at[pl.ds(base_r + k * CD, CD)])
    plsc.subcore_barrier()

    def scat(j, carry):
        pltpu.sync_copy(w2.at[j], acc.at[src2.at[j]], add=True)
        return carry

    lax.fori_loop(0, ND, scat, 0)
    plsc.subcore_barrier()
    pltpu.sync_copy(acc.at[pl.ds(base_r, RPT)], degs_ref.at[c, pl.ds(base_r, RPT)])


_deg_call = functools.partial(
    pl.kernel,
    out_type=jax.ShapeDtypeStruct((NC, NP), jnp.float32),
    mesh=_mesh,
    scratch_types=[
        pltpu.VMEM((ND, CD), jnp.int32),
        pltpu.VMEM((ND, CD), jnp.int32),
        pltpu.VMEM((ND, CD), jnp.float32),
        pltpu.VMEM((CD,), jnp.float32),
        pltpu.VMEM_SHARED((NP,), jnp.float32),
    ],
)(_deg_body)


def _prop_body(g_ref, edge_ref, parts_ref, srcb, dstb, dstm, rows0, rows1, acc,
               semi, semg0, semg1, semc0, semc1):
    c = lax.axis_index("c")
    s = lax.axis_index("s")
    lo = c * HALF

    def idx_start(j, slot):
        pltpu.async_copy(edge_ref.at[0, s, j], srcb.at[slot], semi)
        pltpu.async_copy(edge_ref.at[1, s, j], dstb.at[slot], semi)

    def idx_wait(slot):
        pltpu.make_async_copy(edge_ref.at[0, 0, 0], srcb.at[slot], semi).wait()
        pltpu.make_async_copy(edge_ref.at[0, 0, 0], dstb.at[slot], semi).wait()

    def remap(slot):
        for q in range(CP // 16):
            sv = srcb[slot, pl.ds(q * 16, 16)]
            dv = dstb[slot, pl.ds(q * 16, 16)]
            loc = dv - lo
            bad = jnp.logical_or(
                jnp.logical_or(loc < 0, loc >= HALF), sv == dv
            )
            trash = TRASHP + jnp.bitwise_and(dv, 511)
            dstm[slot, pl.ds(q * 16, 16)] = jnp.where(bad, trash, loc)

    def gstart(slot, buf, sem):
        pltpu.async_copy(g_ref.at[srcb.at[slot]], buf, sem)

    def gwait(buf, sem):
        pltpu.make_async_copy(g_ref.at[pl.ds(0, CP)], buf, sem).wait()

    def scstart(slot, buf, sem):
        pltpu.async_copy(buf, acc.at[dstm.at[slot]], sem, add=True)

    def scwait(slot, buf, sem):
        pltpu.make_async_copy(buf, acc.at[dstm.at[slot]], sem).wait()

    def zrow(r, carry):
        for q in range(F // 16):
            rows0[r, pl.ds(q * 16, 16)] = jnp.zeros((16,), jnp.float32)
        return carry

    lax.fori_loop(0, CP, zrow, 0)
    base_r = s * RPH
    pltpu.sync_copy(rows0, acc.at[pl.ds(base_r, CP)])
    pltpu.sync_copy(rows0, acc.at[pl.ds(base_r + CP, CP)])
    pltpu.sync_copy(rows0.at[pl.ds(0, RPH - 2 * CP)],
                    acc.at[pl.ds(base_r + 2 * CP, RPH - 2 * CP)])
    plsc.subcore_barrier()

    idx_start(0, 0)
    idx_wait(0)
    remap(0)
    gstart(0, rows0, semg0)
    idx_start(1, 1)

    def body(j, carry):
        s0 = lax.rem(j, 2)
        s1 = lax.rem(j + 1, 2)
        even = s0 == 0

        @pl.when(jnp.logical_and(j >= 1, even))
        def _():
            scwait(1, rows1, semc1)

        @pl.when(jnp.logical_and(j >= 1, jnp.logical_not(even)))
        def _():
            scwait(0, rows0, semc0)

        @pl.when(j + 1 < NT)
        def _():
            idx_wait(s1)
            remap(s1)

            @pl.when(even)
            def _():
                gstart(s1, rows1, semg1)

            @pl.when(jnp.logical_not(even))
            def _():
                gstart(s1, rows0, semg0)

        @pl.when(even)
        def _():
            gwait(rows0, semg0)

        @pl.when(jnp.logical_not(even))
        def _():
            gwait(rows1, semg1)

        @pl.when(j + 2 < NT)
        def _():
            idx_start(j + 2, s0)

        @pl.when(even)
        def _():
            scstart(0, rows0, semc0)

        @pl.when(jnp.logical_not(even))
        def _():
            scstart(1, rows1, semc1)

        return carry

    lax.fori_loop(0, NT, body, 0)
    scwait(1, rows1, semc1)
    plsc.subcore_barrier()
    pltpu.sync_copy(acc.at[pl.ds(base_r, RPH)], parts_ref.at[c, pl.ds(base_r, RPH)])


_prop = functools.partial(
    pl.kernel,
    out_type=jax.ShapeDtypeStruct((NC, NPH, F), jnp.float32),
    mesh=_mesh,
    scratch_types=[
        pltpu.VMEM((2, CP), jnp.int32),
        pltpu.VMEM((2, CP), jnp.int32),
        pltpu.VMEM((2, CP), jnp.int32),
        pltpu.VMEM((CP, F), jnp.float32),
        pltpu.VMEM((CP, F), jnp.float32),
        pltpu.VMEM_SHARED((NPH, F), jnp.float32),
        pltpu.SemaphoreType.DMA,
        pltpu.SemaphoreType.DMA,
        pltpu.SemaphoreType.DMA,
        pltpu.SemaphoreType.DMA,
        pltpu.SemaphoreType.DMA,
    ],
)(_prop_body)


_BR = 200
_GRID = N // _BR
_BPH = HALF // _BR

_ROWS_F = pl.BlockSpec((_BR, F), lambda i: (i, 0))
_PARTS = pl.BlockSpec((1, _BR, F), lambda i: (i // _BPH, i % _BPH, 0))
_F_SHAPE = jax.ShapeDtypeStruct((N, F), jnp.float32)


def _tca_body(degp_ref, x_ref, d_ref, g_ref):
    deg = degp_ref[0] + degp_ref[1]
    dis = jnp.where(deg > 0.0, lax.rsqrt(jnp.maximum(deg, 1e-12)), 0.0)
    dmat = jnp.broadcast_to(dis, (_BR, F))
    d_ref[...] = dmat
    g_ref[...] = dmat * x_ref[...]


def _tca(degp, x):
    return pl.pallas_call(
        _tca_body,
        grid=(_GRID,),
        in_specs=[
            pl.BlockSpec((NC, _BR, 1), lambda i: (0, i, 0)),
            _ROWS_F,
        ],
        out_specs=[_ROWS_F, _ROWS_F],
        out_shape=[_F_SHAPE, _F_SHAPE],
    )(degp, x)


def _tcb_body(p_ref, d_ref, tx_ref, g_ref):
    tx = -(d_ref[...] * p_ref[0])
    tx_ref[...] = tx
    g_ref[...] = d_ref[...] * tx


def _tcb(parts, d):
    return pl.pallas_call(
        _tcb_body,
        grid=(_GRID,),
        in_specs=[_PARTS, _ROWS_F],
        out_specs=[_ROWS_F, _ROWS_F],
        out_shape=[_F_SHAPE, _F_SHAPE],
    )(parts, d)


def _tcc1_body(x_ref, t1_ref, p_ref, d_ref, w_ref, b_ref, h_ref, g_ref):
    r = -2.0 * d_ref[...] * p_ref[0]
    acc = jnp.dot(x_ref[...], w_ref[0] - w_ref[2],
                  preferred_element_type=jnp.float32)
    acc += jnp.dot(t1_ref[...], w_ref[1], preferred_element_type=jnp.float32)
    acc += jnp.dot(r, w_ref[2], preferred_element_type=jnp.float32)
    acc += b_ref[...]
    h = jnp.maximum(acc, 0.0)
    h_ref[...] = h
    g_ref[...] = d_ref[...] * h


def _tcc1(x, t1, parts, d, w, b):
    return pl.pallas_call(
        _tcc1_body,
        grid=(_GRID,),
        in_specs=[
            _ROWS_F,
            _ROWS_F,
            _PARTS,
            _ROWS_F,
            pl.BlockSpec((3, F, F), lambda i: (0, 0, 0)),
            pl.BlockSpec((1, F), lambda i: (0, 0)),
        ],
        out_specs=[_ROWS_F, _ROWS_F],
        out_shape=[_F_SHAPE, _F_SHAPE],
    )(x, t1, parts, d, w, b)


def _tcc2_body(h_ref, t1_ref, p_ref, d_ref, w_ref, b_ref, o_ref):
    r = -2.0 * d_ref[...] * p_ref[0]
    acc = jnp.dot(h_ref[...], w_ref[0] - w_ref[2],
                  preferred_element_type=jnp.float32)
    acc += jnp.dot(t1_ref[...], w_ref[1], preferred_element_type=jnp.float32)
    acc += jnp.dot(r, w_ref[2], preferred_element_type=jnp.float32)
    acc += b_ref[...]
    m = jnp.max(acc, axis=1, keepdims=True)
    ex = jnp.exp(acc - m)
    lse = jnp.log(jnp.sum(ex, axis=1, keepdims=True)) + m
    o_ref[...] = acc - lse


def _tcc2(h, t1, parts, d, w, b):
    return pl.pallas_call(
        _tcc2_body,
        grid=(_GRID,),
        in_specs=[
            _ROWS_F,
            _ROWS_F,
            _PARTS,
            _ROWS_F,
            pl.BlockSpec((3, F, C), lambda i: (0, 0, 0)),
            pl.BlockSpec((1, C), lambda i: (0, 0)),
        ],
        out_specs=pl.BlockSpec((_BR, C), lambda i: (i, 0)),
        out_shape=jax.ShapeDtypeStruct((N, C), jnp.float32),
    )(h, t1, parts, d, w, b)


def kernel(x, edge_index, W1, b1, W2, b2):
    ei = jnp.pad(edge_index, ((0, 0), (0, EPAD - E)))
    ei_deg = ei.reshape(2, NW, ND, CD)
    ei_prop = ei.reshape(2, NS, NT, CP)
    degp = _deg_call(ei_deg)
    d, g0 = _tca(degp.reshape(NC, NP, 1), x)
    p = _prop(g0, ei_prop)
    tx1, g1 = _tcb(p, d)
    q = _prop(g1, ei_prop)
    h, gh = _tcc1(x, tx1, q, d, W1, b1.reshape(1, F))
    r1 = _prop(gh, ei_prop)
    ty1, g3 = _tcb(r1, d)
    z = _prop(g3, ei_prop)
    return _tcc2(h, ty1, z, d, W2, b2.reshape(1, C))


# speedup vs baseline: 8.5754x; 1.6668x over previous
import functools

import jax
import jax.numpy as jnp
from jax import lax
from jax.experimental import pallas as pl
from jax.experimental.pallas import tpu as pltpu
from jax.experimental.pallas import tpu_sc as plsc

N = 10000
E = 320000
F = 128
C = 64

NC = 2
NS = 16
NW = NC * NS

CD = 64
ND = 158
EPT = ND * CD
EPAD = EPT * NW
CP = 128
NT = EPAD // NS // CP

NP = 10240
RPT = NP // NS

HALF = N // NC
NPH = 5632
RPH = NPH // NS
TRASHD = N
TRASHP = HALF

_mesh = plsc.VectorSubcoreMesh(
    core_axis_name="c", subcore_axis_name="s", num_cores=NC, num_subcores=NS
)


def _deg_body(edge_ref, degs_ref, src2, dst2, w2, zb, acc):
    c = lax.axis_index("c")
    s = lax.axis_index("s")
    wid = s * NC + c
    pltpu.sync_copy(edge_ref.at[0, wid], src2)
    pltpu.sync_copy(edge_ref.at[1, wid], dst2)

    def prep(j, carry):
        for q in range(CD // 16):
            sv = src2[j, pl.ds(q * 16, 16)]
            dv = dst2[j, pl.ds(q * 16, 16)]
            w2[j, pl.ds(q * 16, 16)] = jnp.where(sv == dv, 0.0, 1.0).astype(
                jnp.float32
            )
        return carry

    lax.fori_loop(0, ND, prep, 0)

    for q in range(CD // 16):
        zb[pl.ds(q * 16, 16)] = jnp.zeros((16,), jnp.float32)
    base_r = s * RPT
    for k in range(RPT // CD):
        pltpu.sync_copy(zb, acc.at[pl.ds(base_r + k * CD, CD)])
    plsc.subcore_barrier()

    def scat(j, carry):
        pltpu.sync_copy(w2.at[j], acc.at[src2.at[j]], add=True)
        return carry

    lax.fori_loop(0, ND, scat, 0)
    plsc.subcore_barrier()
    pltpu.sync_copy(acc.at[pl.ds(base_r, RPT)], degs_ref.at[c, pl.ds(base_r, RPT)])


_deg_call = functools.partial(
    pl.kernel,
    out_type=jax.ShapeDtypeStruct((NC, NP), jnp.float32),
    mesh=_mesh,
    scratch_types=[
        pltpu.VMEM((ND, CD), jnp.int32),
        pltpu.VMEM((ND, CD), jnp.int32),
        pltpu.VMEM((ND, CD), jnp.float32),
        pltpu.VMEM((CD,), jnp.float32),
        pltpu.VMEM_SHARED((NP,), jnp.float32),
    ],
)(_deg_body)


def _prep_body(edge_ref, bsrc_ref, bdst_ref, cnt_ref,
               src1, dst1, os0, od0, os1, od1, cbuf):
    cidx = lax.axis_index("c")
    s = lax.axis_index("s")
    wid = s * NC + cidx
    pltpu.sync_copy(edge_ref.at[0, wid], src1)
    pltpu.sync_copy(edge_ref.at[1, wid], dst1)

    zer = jnp.zeros((16,), jnp.int32)
    tra = jnp.full((16,), TRASHP, jnp.int32)

    def pre(i, carry):
        os0[pl.ds(i * 16, 16)] = zer
        od0[pl.ds(i * 16, 16)] = tra
        os1[pl.ds(i * 16, 16)] = zer
        od1[pl.ds(i * 16, 16)] = tra
        return carry

    lax.fori_loop(0, (EPT + 16) // 16, pre, 0)

    one = jnp.full((16,), 1, jnp.int32)
    zero = jnp.full((16,), 0, jnp.int32)
    dump = jnp.full((16,), EPT, jnp.int32)

    def grp(j, carry):
        off0, off1 = carry
        for q in range(CD // 16):
            sv = src1[pl.ds(j * CD + q * 16, 16)]
            dv = dst1[pl.ds(j * CD + q * 16, 16)]
            valid = sv != dv
            is0 = dv < HALF
            m0 = jnp.logical_and(valid, is0)
            m1 = jnp.logical_and(valid, jnp.logical_not(is0))
            mc0 = jnp.where(m0, one, zero)
            mc1 = jnp.where(m1, one, zero)
            ps0 = plsc.cumsum(mc0)
            ps1 = plsc.cumsum(mc1)
            i0 = jnp.where(m0, off0 + ps0 - mc0, dump)
            i1 = jnp.where(m1, off1 + ps1 - mc1, dump)
            plsc.store_scatter(os0, [i0], sv)
            plsc.store_scatter(od0, [i0], dv)
            plsc.store_scatter(os1, [i1], sv)
            plsc.store_scatter(od1, [i1], dv - HALF)
            off0 = off0 + plsc.all_reduce_population_count(m0)
            off1 = off1 + plsc.all_reduce_population_count(m1)
        return (off0, off1)

    off0, off1 = lax.fori_loop(0, ND, grp, (zero, zero))

    pltpu.sync_copy(os0.at[pl.ds(0, EPT)], bsrc_ref.at[0, wid])
    pltpu.sync_copy(od0.at[pl.ds(0, EPT)], bdst_ref.at[0, wid])
    pltpu.sync_copy(os1.at[pl.ds(0, EPT)], bsrc_ref.at[1, wid])
    pltpu.sync_copy(od1.at[pl.ds(0, EPT)], bdst_ref.at[1, wid])

    cpv = jnp.full((16,), CP, jnp.int32)
    nch0 = lax.div(off0 + (CP - 1), cpv)
    nch1 = lax.div(off1 + (CP - 1), cpv)
    iot = lax.iota(jnp.int32, 16)
    cbuf[...] = jnp.where(iot == 0, nch0, jnp.where(iot == 1, nch1, zero))
    pltpu.sync_copy(cbuf, cnt_ref.at[wid])


_prep_call = functools.partial(
    pl.kernel,
    out_type=(
        jax.ShapeDtypeStruct((NC, NW, EPT), jnp.int32),
        jax.ShapeDtypeStruct((NC, NW, EPT), jnp.int32),
        jax.ShapeDtypeStruct((NW, 16), jnp.int32),
    ),
    mesh=_mesh,
    compiler_params=pltpu.CompilerParams(needs_layout_passes=False),
    scratch_types=[
        pltpu.VMEM((EPT,), jnp.int32),
        pltpu.VMEM((EPT,), jnp.int32),
        pltpu.VMEM((EPT + 16,), jnp.int32),
        pltpu.VMEM((EPT + 16,), jnp.int32),
        pltpu.VMEM((EPT + 16,), jnp.int32),
        pltpu.VMEM((EPT + 16,), jnp.int32),
        pltpu.VMEM((16,), jnp.int32),
    ],
)(_prep_body)


def _prop_body(g_ref, bsrc_ref, bdst_ref, cnt_ref, parts_ref,
               srcb, dstm, rows0, rows1, cbuf, acc,
               semi, semg0, semg1, semc0, semc1):
    c = lax.axis_index("c")
    s = lax.axis_index("s")
    w0 = 2 * s
    w1 = 2 * s + 1

    pltpu.sync_copy(cnt_ref.at[w0], cbuf)
    cb0 = cbuf[...]
    pltpu.sync_copy(cnt_ref.at[w1], cbuf)
    cb1 = cbuf[...]
    n0 = jnp.where(c == 0, cb0[0], cb0[1])
    n1 = jnp.where(c == 0, cb1[0], cb1[1])
    ntot = n0 + n1

    def idx_start(j, slot):
        @pl.when(j < n0)
        def _():
            pltpu.async_copy(bsrc_ref.at[c, w0, pl.ds(j * CP, CP)],
                             srcb.at[slot], semi)
            pltpu.async_copy(bdst_ref.at[c, w0, pl.ds(j * CP, CP)],
                             dstm.at[slot], semi)

        @pl.when(j >= n0)
        def _():
            pltpu.async_copy(bsrc_ref.at[c, w1, pl.ds((j - n0) * CP, CP)],
                             srcb.at[slot], semi)
            pltpu.async_copy(bdst_ref.at[c, w1, pl.ds((j - n0) * CP, CP)],
                             dstm.at[slot], semi)

    def idx_wait(slot):
        pltpu.make_async_copy(bsrc_ref.at[0, 0, pl.ds(0, CP)],
                              srcb.at[slot], semi).wait()
        pltpu.make_async_copy(bsrc_ref.at[0, 0, pl.ds(0, CP)],
                              dstm.at[slot], semi).wait()

    def gstart(slot, buf, sem):
        pltpu.async_copy(g_ref.at[srcb.at[slot]], buf, sem)

    def gwait(buf, sem):
        pltpu.make_async_copy(g_ref.at[pl.ds(0, CP)], buf, sem).wait()

    def scstart(slot, buf, sem):
        pltpu.async_copy(buf, acc.at[dstm.at[slot]], sem, add=True)

    def scwait(slot, buf, sem):
        pltpu.make_async_copy(buf, acc.at[dstm.at[slot]], sem).wait()

    def zrow(r, carry):
        for q in range(F // 16):
            rows0[r, pl.ds(q * 16, 16)] = jnp.zeros((16,), jnp.float32)
        return carry

    lax.fori_loop(0, CP, zrow, 0)
    base_r = s * RPH
    pltpu.sync_copy(rows0, acc.at[pl.ds(base_r, CP)])
    pltpu.sync_copy(rows0, acc.at[pl.ds(base_r + CP, CP)])
    pltpu.sync_copy(rows0.at[pl.ds(0, RPH - 2 * CP)],
                    acc.at[pl.ds(base_r + 2 * CP, RPH - 2 * CP)])
    plsc.subcore_barrier()

    @pl.when(ntot > 0)
    def _():
        idx_start(0, 0)
        idx_wait(0)
        gstart(0, rows0, semg0)

        @pl.when(ntot > 1)
        def _():
            idx_start(1, 1)

        def body(j, carry):
            s0 = lax.rem(j, 2)
            s1 = lax.rem(j + 1, 2)
            even = s0 == 0

            @pl.when(jnp.logical_and(j >= 1, even))
            def _():
                scwait(1, rows1, semc1)

            @pl.when(jnp.logical_and(j >= 1, jnp.logical_not(even)))
            def _():
                scwait(0, rows0, semc0)

            @pl.when(j + 1 < ntot)
            def _():
                idx_wait(s1)

                @pl.when(even)
                def _():
                    gstart(s1, rows1, semg1)

                @pl.when(jnp.logical_not(even))
                def _():
                    gstart(s1, rows0, semg0)

            @pl.when(even)
            def _():
                gwait(rows0, semg0)

            @pl.when(jnp.logical_not(even))
            def _():
                gwait(rows1, semg1)

            @pl.when(j + 2 < ntot)
            def _():
                idx_start(j + 2, s0)

            @pl.when(even)
            def _():
                scstart(0, rows0, semc0)

            @pl.when(jnp.logical_not(even))
            def _():
                scstart(1, rows1, semc1)

            return carry

        lax.fori_loop(0, ntot, body, 0)
        last_odd = lax.rem(ntot - 1, 2) == 1

        @pl.when(last_odd)
        def _():
            scwait(1, rows1, semc1)

        @pl.when(jnp.logical_not(last_odd))
        def _():
            scwait(0, rows0, semc0)

    plsc.subcore_barrier()
    pltpu.sync_copy(acc.at[pl.ds(base_r, RPH)], parts_ref.at[c, pl.ds(base_r, RPH)])


_prop = functools.partial(
    pl.kernel,
    out_type=jax.ShapeDtypeStruct((NC, NPH, F), jnp.float32),
    mesh=_mesh,
    scratch_types=[
        pltpu.VMEM((2, CP), jnp.int32),
        pltpu.VMEM((2, CP), jnp.int32),
        pltpu.VMEM((CP, F), jnp.float32),
        pltpu.VMEM((CP, F), jnp.float32),
        pltpu.VMEM((16,), jnp.int32),
        pltpu.VMEM_SHARED((NPH, F), jnp.float32),
        pltpu.SemaphoreType.DMA,
        pltpu.SemaphoreType.DMA,
        pltpu.SemaphoreType.DMA,
        pltpu.SemaphoreType.DMA,
        pltpu.SemaphoreType.DMA,
    ],
)(_prop_body)


_BR = 200
_GRID = N // _BR
_BPH = HALF // _BR

_ROWS_F = pl.BlockSpec((_BR, F), lambda i: (i, 0))
_PARTS = pl.BlockSpec((1, _BR, F), lambda i: (i // _BPH, i % _BPH, 0))
_F_SHAPE = jax.ShapeDtypeStruct((N, F), jnp.float32)


def _tca_body(degp_ref, x_ref, d_ref, g_ref):
    deg = degp_ref[0] + degp_ref[1]
    dis = jnp.where(deg > 0.0, lax.rsqrt(jnp.maximum(deg, 1e-12)), 0.0)
    dmat = jnp.broadcast_to(dis, (_BR, F))
    d_ref[...] = dmat
    g_ref[...] = dmat * x_ref[...]


def _tca(degp, x):
    return pl.pallas_call(
        _tca_body,
        grid=(_GRID,),
        in_specs=[
            pl.BlockSpec((NC, _BR, 1), lambda i: (0, i, 0)),
            _ROWS_F,
        ],
        out_specs=[_ROWS_F, _ROWS_F],
        out_shape=[_F_SHAPE, _F_SHAPE],
    )(degp, x)


def _tcb_body(p_ref, d_ref, tx_ref, g_ref):
    tx = -(d_ref[...] * p_ref[0])
    tx_ref[...] = tx
    g_ref[...] = d_ref[...] * tx


def _tcb(parts, d):
    return pl.pallas_call(
        _tcb_body,
        grid=(_GRID,),
        in_specs=[_PARTS, _ROWS_F],
        out_specs=[_ROWS_F, _ROWS_F],
        out_shape=[_F_SHAPE, _F_SHAPE],
    )(parts, d)


def _tcc1_body(x_ref, t1_ref, p_ref, d_ref, w_ref, b_ref, h_ref, g_ref):
    r = -2.0 * d_ref[...] * p_ref[0]
    acc = jnp.dot(x_ref[...], w_ref[0] - w_ref[2],
                  preferred_element_type=jnp.float32)
    acc += jnp.dot(t1_ref[...], w_ref[1], preferred_element_type=jnp.float32)
    acc += jnp.dot(r, w_ref[2], preferred_element_type=jnp.float32)
    acc += b_ref[...]
    h = jnp.maximum(acc, 0.0)
    h_ref[...] = h
    g_ref[...] = d_ref[...] * h


def _tcc1(x, t1, parts, d, w, b):
    return pl.pallas_call(
        _tcc1_body,
        grid=(_GRID,),
        in_specs=[
            _ROWS_F,
            _ROWS_F,
            _PARTS,
            _ROWS_F,
            pl.BlockSpec((3, F, F), lambda i: (0, 0, 0)),
            pl.BlockSpec((1, F), lambda i: (0, 0)),
        ],
        out_specs=[_ROWS_F, _ROWS_F],
        out_shape=[_F_SHAPE, _F_SHAPE],
    )(x, t1, parts, d, w, b)


def _tcc2_body(h_ref, t1_ref, p_ref, d_ref, w_ref, b_ref, o_ref):
    r = -2.0 * d_ref[...] * p_ref[0]
    acc = jnp.dot(h_ref[...], w_ref[0] - w_ref[2],
                  preferred_element_type=jnp.float32)
    acc += jnp.dot(t1_ref[...], w_ref[1], preferred_element_type=jnp.float32)
    acc += jnp.dot(r, w_ref[2], preferred_element_type=jnp.float32)
    acc += b_ref[...]
    m = jnp.max(acc, axis=1, keepdims=True)
    ex = jnp.exp(acc - m)
    lse = jnp.log(jnp.sum(ex, axis=1, keepdims=True)) + m
    o_ref[...] = acc - lse


def _tcc2(h, t1, parts, d, w, b):
    return pl.pallas_call(
        _tcc2_body,
        grid=(_GRID,),
        in_specs=[
            _ROWS_F,
            _ROWS_F,
            _PARTS,
            _ROWS_F,
            pl.BlockSpec((3, F, C), lambda i: (0, 0, 0)),
            pl.BlockSpec((1, C), lambda i: (0, 0)),
        ],
        out_specs=pl.BlockSpec((_BR, C), lambda i: (i, 0)),
        out_shape=jax.ShapeDtypeStruct((N, C), jnp.float32),
    )(h, t1, parts, d, w, b)


def kernel(x, edge_index, W1, b1, W2, b2):
    ei = jnp.pad(edge_index, ((0, 0), (0, EPAD - E)))
    ei_deg = ei.reshape(2, NW, ND, CD)
    degp = _deg_call(ei_deg)
    bsrc, bdst, cnt = _prep_call(ei.reshape(2, NW, EPT))
    d, g0 = _tca(degp.reshape(NC, NP, 1), x)
    p = _prop(g0, bsrc, bdst, cnt)
    tx1, g1 = _tcb(p, d)
    q = _prop(g1, bsrc, bdst, cnt)
    h, gh = _tcc1(x, tx1, q, d, W1, b1.reshape(1, F))
    r1 = _prop(gh, bsrc, bdst, cnt)
    ty1, g3 = _tcb(r1, d)
    z = _prop(g3, bsrc, bdst, cnt)
    return _tcc2(h, ty1, z, d, W2, b2.reshape(1, C))


# speedup vs baseline: 9.8371x; 1.1471x over previous
import functools

import jax
import jax.numpy as jnp
from jax import lax
from jax.experimental import pallas as pl
from jax.experimental.pallas import tpu as pltpu
from jax.experimental.pallas import tpu_sc as plsc

N = 10000
E = 320000
F = 128
C = 64

NC = 2
NS = 16
NW = NC * NS

CD = 64
ND = 158
EPT = ND * CD
EPAD = EPT * NW
CP = 64
NBLK = EPT // CP
DBL = 2 * EPT
DEPTH = 4
IDEPTH = 8

NP = 10240
RPT = NP // NS

HALF = N // NC
NPH = 5632
RPH = NPH // NS
TRASHD = N
TRASHP = HALF

_mesh = plsc.VectorSubcoreMesh(
    core_axis_name="c", subcore_axis_name="s", num_cores=NC, num_subcores=NS
)


def _deg_body(edge_ref, degs_ref, src2, dst2, w2, zb, acc):
    c = lax.axis_index("c")
    s = lax.axis_index("s")
    wid = s * NC + c
    pltpu.sync_copy(edge_ref.at[0, wid], src2)
    pltpu.sync_copy(edge_ref.at[1, wid], dst2)

    def prep(j, carry):
        for q in range(CD // 16):
            sv = src2[j, pl.ds(q * 16, 16)]
            dv = dst2[j, pl.ds(q * 16, 16)]
            w2[j, pl.ds(q * 16, 16)] = jnp.where(sv == dv, 0.0, 1.0).astype(
                jnp.float32
            )
        return carry

    lax.fori_loop(0, ND, prep, 0)

    for q in range(CD // 16):
        zb[pl.ds(q * 16, 16)] = jnp.zeros((16,), jnp.float32)
    base_r = s * RPT
    for k in range(RPT // CD):
        pltpu.sync_copy(zb, acc.at[pl.ds(base_r + k * CD, CD)])
    plsc.subcore_barrier()

    def scat(j, carry):
        pltpu.sync_copy(w2.at[j], acc.at[src2.at[j]], add=True)
        return carry

    lax.fori_loop(0, ND, scat, 0)
    plsc.subcore_barrier()
    pltpu.sync_copy(acc.at[pl.ds(base_r, RPT)], degs_ref.at[c, pl.ds(base_r, RPT)])


_deg_call = functools.partial(
    pl.kernel,
    out_type=jax.ShapeDtypeStruct((NC, NP), jnp.float32),
    mesh=_mesh,
    scratch_types=[
        pltpu.VMEM((ND, CD), jnp.int32),
        pltpu.VMEM((ND, CD), jnp.int32),
        pltpu.VMEM((ND, CD), jnp.float32),
        pltpu.VMEM((CD,), jnp.float32),
        pltpu.VMEM_SHARED((NP,), jnp.float32),
    ],
)(_deg_body)


def _prep_body(edge_ref, bint_ref, cnt_ref,
               src1, dst1, ob0, ob1, cbuf):
    cidx = lax.axis_index("c")
    s = lax.axis_index("s")
    wid = s * NC + cidx
    pltpu.sync_copy(edge_ref.at[0, wid], src1)
    pltpu.sync_copy(edge_ref.at[1, wid], dst1)

    tra = jnp.full((16,), TRASHP, jnp.int32)

    def pre(i, carry):
        ob0[pl.ds(i * 16, 16)] = tra
        ob1[pl.ds(i * 16, 16)] = tra
        return carry

    lax.fori_loop(0, (DBL + 128) // 16, pre, 0)

    one = jnp.full((16,), 1, jnp.int32)
    zero = jnp.full((16,), 0, jnp.int32)
    dump = jnp.full((16,), DBL, jnp.int32)

    def grp(j, carry):
        off0, off1 = carry
        for q in range(CD // 16):
            sv = src1[pl.ds(j * CD + q * 16, 16)]
            dv = dst1[pl.ds(j * CD + q * 16, 16)]
            valid = sv != dv
            is0 = dv < HALF
            m0 = jnp.logical_and(valid, is0)
            m1 = jnp.logical_and(valid, jnp.logical_not(is0))
            mc0 = jnp.where(m0, one, zero)
            mc1 = jnp.where(m1, one, zero)
            ps0 = plsc.cumsum(mc0)
            ps1 = plsc.cumsum(mc1)
            e0 = off0 + ps0 - mc0
            e1 = off1 + ps1 - mc1
            p0 = ((e0 >> 6) << 7) + (e0 & 63)
            p1 = ((e1 >> 6) << 7) + (e1 & 63)
            i0 = jnp.where(m0, p0, dump)
            i1 = jnp.where(m1, p1, dump)
            plsc.store_scatter(ob0, [i0], sv)
            plsc.store_scatter(ob0, [i0 + CP], dv)
            plsc.store_scatter(ob1, [i1], sv)
            plsc.store_scatter(ob1, [i1 + CP], dv - HALF)
            off0 = off0 + plsc.all_reduce_population_count(m0)
            off1 = off1 + plsc.all_reduce_population_count(m1)
        return (off0, off1)

    off0, off1 = lax.fori_loop(0, ND, grp, (zero, zero))

    pltpu.sync_copy(ob0.at[pl.ds(0, DBL)], bint_ref.at[0, wid])
    pltpu.sync_copy(ob1.at[pl.ds(0, DBL)], bint_ref.at[1, wid])

    nch0 = (off0 + (CP - 1)) >> 6
    nch1 = (off1 + (CP - 1)) >> 6
    iot = lax.iota(jnp.int32, 16)
    cbuf[...] = jnp.where(iot == 0, nch0, jnp.where(iot == 1, nch1, zero))
    pltpu.sync_copy(cbuf, cnt_ref.at[wid])


_prep_call = functools.partial(
    pl.kernel,
    out_type=(
        jax.ShapeDtypeStruct((NC, NW, DBL), jnp.int32),
        jax.ShapeDtypeStruct((NW, 16), jnp.int32),
    ),
    mesh=_mesh,
    compiler_params=pltpu.CompilerParams(needs_layout_passes=False),
    scratch_types=[
        pltpu.VMEM((EPT,), jnp.int32),
        pltpu.VMEM((EPT,), jnp.int32),
        pltpu.VMEM((DBL + 128,), jnp.int32),
        pltpu.VMEM((DBL + 128,), jnp.int32),
        pltpu.VMEM((16,), jnp.int32),
    ],
)(_prep_body)


def _prop_body(g_ref, bint_ref, cnt_ref, parts_ref,
               sd, rows0, rows1, rows2, rows3, cbuf, acc,
               semi, semg0, semg1, semg2, semg3, semc0, semc1, semc2, semc3):
    c = lax.axis_index("c")
    s = lax.axis_index("s")
    w0 = 2 * s
    w1 = 2 * s + 1

    pltpu.sync_copy(cnt_ref.at[w0], cbuf)
    cb0 = cbuf[...]
    pltpu.sync_copy(cnt_ref.at[w1], cbuf)
    cb1 = cbuf[...]
    n0 = jnp.where(c == 0, cb0[0], cb0[1])
    n1 = jnp.where(c == 0, cb1[0], cb1[1])
    ntot = n0 + n1

    rows = [rows0, rows1, rows2, rows3]
    semg = [semg0, semg1, semg2, semg3]
    semc = [semc0, semc1, semc2, semc3]

    def idx_start(j):
        jj = lax.rem(j, IDEPTH)

        @pl.when(j < n0)
        def _():
            pltpu.async_copy(bint_ref.at[c, w0, j], sd.at[jj], semi)

        @pl.when(j >= n0)
        def _():
            pltpu.async_copy(bint_ref.at[c, w1, j - n0], sd.at[jj], semi)

    def idx_wait(j):
        jj = lax.rem(j, IDEPTH)
        pltpu.make_async_copy(bint_ref.at[0, 0, 0], sd.at[jj], semi).wait()

    def gstart(j, k):
        jj = lax.rem(j, IDEPTH)
        pltpu.async_copy(g_ref.at[sd.at[jj, 0]], rows[k], semg[k])

    def gwait(k):
        pltpu.make_async_copy(g_ref.at[pl.ds(0, CP)], rows[k], semg[k]).wait()

    def scstart(j, k):
        jj = lax.rem(j, IDEPTH)
        pltpu.async_copy(rows[k], acc.at[sd.at[jj, 1]], semc[k], add=True)

    def scwait(k):
        pltpu.make_async_copy(rows[k], acc.at[sd.at[0, 1]], semc[k]).wait()

    def zrow(r, carry):
        for q in range(F // 16):
            rows0[r, pl.ds(q * 16, 16)] = jnp.zeros((16,), jnp.float32)
        return carry

    lax.fori_loop(0, CP, zrow, 0)
    base_r = s * RPH
    for t in range(RPH // CP):
        pltpu.sync_copy(rows0, acc.at[pl.ds(base_r + t * CP, CP)])
    pltpu.sync_copy(rows0.at[pl.ds(0, RPH - (RPH // CP) * CP)],
                    acc.at[pl.ds(base_r + (RPH // CP) * CP,
                                 RPH - (RPH // CP) * CP)])
    plsc.subcore_barrier()

    for k in range(DEPTH):
        @pl.when(k < ntot)
        def _(k=k):
            idx_start(k)

    jm = (ntot + (DEPTH - 1)) >> 2

    def body(J, carry):
        b = J * DEPTH
        for k in range(DEPTH):
            @pl.when(b + k - DEPTH >= 0)
            def _(k=k):
                scwait(k)
        for k in range(DEPTH):
            j = b + k

            @pl.when(j < ntot)
            def _(j=j, k=k):
                idx_wait(j)
                gstart(j, k)
        for k in range(DEPTH):
            j = b + k

            @pl.when(j + DEPTH < ntot)
            def _(j=j):
                idx_start(j + DEPTH)
        for k in range(DEPTH):
            j = b + k

            @pl.when(j < ntot)
            def _(j=j, k=k):
                gwait(k)
                scstart(j, k)
        return carry

    lax.fori_loop(0, jm, body, 0)
    base = (jm - 1) * DEPTH
    for k in range(DEPTH):
        @pl.when(jnp.logical_and(base + k >= 0, base + k < ntot))
        def _(k=k):
            scwait(k)

    plsc.subcore_barrier()
    pltpu.sync_copy(acc.at[pl.ds(base_r, RPH)], parts_ref.at[c, pl.ds(base_r, RPH)])


_prop = functools.partial(
    pl.kernel,
    out_type=jax.ShapeDtypeStruct((NC, NPH, F), jnp.float32),
    mesh=_mesh,
    scratch_types=[
        pltpu.VMEM((IDEPTH, 2, CP), jnp.int32),
        pltpu.VMEM((CP, F), jnp.float32),
        pltpu.VMEM((CP, F), jnp.float32),
        pltpu.VMEM((CP, F), jnp.float32),
        pltpu.VMEM((CP, F), jnp.float32),
        pltpu.VMEM((16,), jnp.int32),
        pltpu.VMEM_SHARED((NPH, F), jnp.float32),
        pltpu.SemaphoreType.DMA,
        pltpu.SemaphoreType.DMA,
        pltpu.SemaphoreType.DMA,
        pltpu.SemaphoreType.DMA,
        pltpu.SemaphoreType.DMA,
        pltpu.SemaphoreType.DMA,
        pltpu.SemaphoreType.DMA,
        pltpu.SemaphoreType.DMA,
        pltpu.SemaphoreType.DMA,
    ],
)(_prop_body)


_BR = 200
_GRID = N // _BR
_BPH = HALF // _BR

_ROWS_F = pl.BlockSpec((_BR, F), lambda i: (i, 0))
_PARTS = pl.BlockSpec((1, _BR, F), lambda i: (i // _BPH, i % _BPH, 0))
_F_SHAPE = jax.ShapeDtypeStruct((N, F), jnp.float32)


def _tca_body(degp_ref, x_ref, d_ref, g_ref):
    deg = degp_ref[0] + degp_ref[1]
    dis = jnp.where(deg > 0.0, lax.rsqrt(jnp.maximum(deg, 1e-12)), 0.0)
    dmat = jnp.broadcast_to(dis, (_BR, F))
    d_ref[...] = dmat
    g_ref[...] = dmat * x_ref[...]


def _tca(degp, x):
    return pl.pallas_call(
        _tca_body,
        grid=(_GRID,),
        in_specs=[
            pl.BlockSpec((NC, _BR, 1), lambda i: (0, i, 0)),
            _ROWS_F,
        ],
        out_specs=[_ROWS_F, _ROWS_F],
        out_shape=[_F_SHAPE, _F_SHAPE],
    )(degp, x)


def _tcb_body(p_ref, d_ref, tx_ref, g_ref):
    tx = -(d_ref[...] * p_ref[0])
    tx_ref[...] = tx
    g_ref[...] = d_ref[...] * tx


def _tcb(parts, d):
    return pl.pallas_call(
        _tcb_body,
        grid=(_GRID,),
        in_specs=[_PARTS, _ROWS_F],
        out_specs=[_ROWS_F, _ROWS_F],
        out_shape=[_F_SHAPE, _F_SHAPE],
    )(parts, d)


def _tcc1_body(x_ref, t1_ref, p_ref, d_ref, w_ref, b_ref, h_ref, g_ref):
    r = -2.0 * d_ref[...] * p_ref[0]
    acc = jnp.dot(x_ref[...], w_ref[0] - w_ref[2],
                  preferred_element_type=jnp.float32)
    acc += jnp.dot(t1_ref[...], w_ref[1], preferred_element_type=jnp.float32)
    acc += jnp.dot(r, w_ref[2], preferred_element_type=jnp.float32)
    acc += b_ref[...]
    h = jnp.maximum(acc, 0.0)
    h_ref[...] = h
    g_ref[...] = d_ref[...] * h


def _tcc1(x, t1, parts, d, w, b):
    return pl.pallas_call(
        _tcc1_body,
        grid=(_GRID,),
        in_specs=[
            _ROWS_F,
            _ROWS_F,
            _PARTS,
            _ROWS_F,
            pl.BlockSpec((3, F, F), lambda i: (0, 0, 0)),
            pl.BlockSpec((1, F), lambda i: (0, 0)),
        ],
        out_specs=[_ROWS_F, _ROWS_F],
        out_shape=[_F_SHAPE, _F_SHAPE],
    )(x, t1, parts, d, w, b)


def _tcc2_body(h_ref, t1_ref, p_ref, d_ref, w_ref, b_ref, o_ref):
    r = -2.0 * d_ref[...] * p_ref[0]
    acc = jnp.dot(h_ref[...], w_ref[0] - w_ref[2],
                  preferred_element_type=jnp.float32)
    acc += jnp.dot(t1_ref[...], w_ref[1], preferred_element_type=jnp.float32)
    acc += jnp.dot(r, w_ref[2], preferred_element_type=jnp.float32)
    acc += b_ref[...]
    m = jnp.max(acc, axis=1, keepdims=True)
    ex = jnp.exp(acc - m)
    lse = jnp.log(jnp.sum(ex, axis=1, keepdims=True)) + m
    o_ref[...] = acc - lse


def _tcc2(h, t1, parts, d, w, b):
    return pl.pallas_call(
        _tcc2_body,
        grid=(_GRID,),
        in_specs=[
            _ROWS_F,
            _ROWS_F,
            _PARTS,
            _ROWS_F,
            pl.BlockSpec((3, F, C), lambda i: (0, 0, 0)),
            pl.BlockSpec((1, C), lambda i: (0, 0)),
        ],
        out_specs=pl.BlockSpec((_BR, C), lambda i: (i, 0)),
        out_shape=jax.ShapeDtypeStruct((N, C), jnp.float32),
    )(h, t1, parts, d, w, b)


def kernel(x, edge_index, W1, b1, W2, b2):
    ei = jnp.pad(edge_index, ((0, 0), (0, EPAD - E)))
    ei_deg = ei.reshape(2, NW, ND, CD)
    degp = _deg_call(ei_deg)
    bint, cnt = _prep_call(ei.reshape(2, NW, EPT))
    bint = bint.reshape(NC, NW, NBLK, 2, CP)
    d, g0 = _tca(degp.reshape(NC, NP, 1), x)
    p = _prop(g0, bint, cnt)
    tx1, g1 = _tcb(p, d)
    q = _prop(g1, bint, cnt)
    h, gh = _tcc1(x, tx1, q, d, W1, b1.reshape(1, F))
    r1 = _prop(gh, bint, cnt)
    ty1, g3 = _tcb(r1, d)
    z = _prop(g3, bint, cnt)
    return _tcc2(h, ty1, z, d, W2, b2.reshape(1, C))


# speedup vs baseline: 11.3130x; 1.1500x over previous
import functools

import jax
import jax.numpy as jnp
from jax import lax
from jax.experimental import pallas as pl
from jax.experimental.pallas import tpu as pltpu
from jax.experimental.pallas import tpu_sc as plsc

N = 10000
E = 320000
F = 128
C = 64

NC = 2
NS = 16
NW = NC * NS

CD = 64
ND = 158
EPT = ND * CD
EPAD = EPT * NW
CP = 32
CPL = 5
NBLK = EPT // CP
DBL = 2 * EPT
DEPTH = 8
DPL = 3
IDEPTH = 16

NP = 10240
RPT = NP // NS

HALF = N // NC
NPH = 5632
RPH = NPH // NS
TRASHD = N
TRASHP = HALF

_mesh = plsc.VectorSubcoreMesh(
    core_axis_name="c", subcore_axis_name="s", num_cores=NC, num_subcores=NS
)


def _deg_body(edge_ref, degs_ref, src2, dst2, w2, zb, acc):
    c = lax.axis_index("c")
    s = lax.axis_index("s")
    wid = s * NC + c
    pltpu.sync_copy(edge_ref.at[0, wid], src2)
    pltpu.sync_copy(edge_ref.at[1, wid], dst2)

    def prep(j, carry):
        for q in range(CD // 16):
            sv = src2[j, pl.ds(q * 16, 16)]
            dv = dst2[j, pl.ds(q * 16, 16)]
            w2[j, pl.ds(q * 16, 16)] = jnp.where(sv == dv, 0.0, 1.0).astype(
                jnp.float32
            )
        return carry

    lax.fori_loop(0, ND, prep, 0)

    for q in range(CD // 16):
        zb[pl.ds(q * 16, 16)] = jnp.zeros((16,), jnp.float32)
    base_r = s * RPT
    for k in range(RPT // CD):
        pltpu.sync_copy(zb, acc.at[pl.ds(base_r + k * CD, CD)])
    plsc.subcore_barrier()

    def scat(j, carry):
        pltpu.sync_copy(w2.at[j], acc.at[src2.at[j]], add=True)
        return carry

    lax.fori_loop(0, ND, scat, 0)
    plsc.subcore_barrier()
    pltpu.sync_copy(acc.at[pl.ds(base_r, RPT)], degs_ref.at[c, pl.ds(base_r, RPT)])


_deg_call = functools.partial(
    pl.kernel,
    out_type=jax.ShapeDtypeStruct((NC, NP), jnp.float32),
    mesh=_mesh,
    scratch_types=[
        pltpu.VMEM((ND, CD), jnp.int32),
        pltpu.VMEM((ND, CD), jnp.int32),
        pltpu.VMEM((ND, CD), jnp.float32),
        pltpu.VMEM((CD,), jnp.float32),
        pltpu.VMEM_SHARED((NP,), jnp.float32),
    ],
)(_deg_body)


def _prep_body(edge_ref, bint_ref, cnt_ref,
               src1, dst1, ob0, ob1, cbuf):
    cidx = lax.axis_index("c")
    s = lax.axis_index("s")
    wid = s * NC + cidx
    pltpu.sync_copy(edge_ref.at[0, wid], src1)
    pltpu.sync_copy(edge_ref.at[1, wid], dst1)

    tra = jnp.full((16,), TRASHP, jnp.int32)

    def pre(i, carry):
        ob0[pl.ds(i * 16, 16)] = tra
        ob1[pl.ds(i * 16, 16)] = tra
        return carry

    lax.fori_loop(0, (DBL + 128) // 16, pre, 0)

    one = jnp.full((16,), 1, jnp.int32)
    zero = jnp.full((16,), 0, jnp.int32)
    dump = jnp.full((16,), DBL, jnp.int32)

    def grp(j, carry):
        off0, off1 = carry
        for q in range(CD // 16):
            sv = src1[pl.ds(j * CD + q * 16, 16)]
            dv = dst1[pl.ds(j * CD + q * 16, 16)]
            valid = sv != dv
            is0 = dv < HALF
            m0 = jnp.logical_and(valid, is0)
            m1 = jnp.logical_and(valid, jnp.logical_not(is0))
            mc0 = jnp.where(m0, one, zero)
            mc1 = jnp.where(m1, one, zero)
            ps0 = plsc.cumsum(mc0)
            ps1 = plsc.cumsum(mc1)
            e0 = off0 + ps0 - mc0
            e1 = off1 + ps1 - mc1
            p0 = ((e0 >> CPL) << (CPL + 1)) + (e0 & (CP - 1))
            p1 = ((e1 >> CPL) << (CPL + 1)) + (e1 & (CP - 1))
            i0 = jnp.where(m0, p0, dump)
            i1 = jnp.where(m1, p1, dump)
            plsc.store_scatter(ob0, [i0], sv)
            plsc.store_scatter(ob0, [i0 + CP], dv)
            plsc.store_scatter(ob1, [i1], sv)
            plsc.store_scatter(ob1, [i1 + CP], dv - HALF)
            off0 = off0 + plsc.all_reduce_population_count(m0)
            off1 = off1 + plsc.all_reduce_population_count(m1)
        return (off0, off1)

    off0, off1 = lax.fori_loop(0, ND, grp, (zero, zero))

    pltpu.sync_copy(ob0.at[pl.ds(0, DBL)], bint_ref.at[0, wid])
    pltpu.sync_copy(ob1.at[pl.ds(0, DBL)], bint_ref.at[1, wid])

    nch0 = (off0 + (CP - 1)) >> CPL
    nch1 = (off1 + (CP - 1)) >> CPL
    iot = lax.iota(jnp.int32, 16)
    cbuf[...] = jnp.where(iot == 0, nch0, jnp.where(iot == 1, nch1, zero))
    pltpu.sync_copy(cbuf, cnt_ref.at[wid])


_prep_call = functools.partial(
    pl.kernel,
    out_type=(
        jax.ShapeDtypeStruct((NC, NW, DBL), jnp.int32),
        jax.ShapeDtypeStruct((NW, 16), jnp.int32),
    ),
    mesh=_mesh,
    compiler_params=pltpu.CompilerParams(needs_layout_passes=False),
    scratch_types=[
        pltpu.VMEM((EPT,), jnp.int32),
        pltpu.VMEM((EPT,), jnp.int32),
        pltpu.VMEM((DBL + 128,), jnp.int32),
        pltpu.VMEM((DBL + 128,), jnp.int32),
        pltpu.VMEM((16,), jnp.int32),
    ],
)(_prep_body)


def _prop_body(g_ref, bint_ref, cnt_ref, parts_ref,
               sd, rows0, rows1, rows2, rows3, rows4, rows5, rows6, rows7,
               cbuf, acc, semi, semg0, semg1, semg2, semg3, semg4, semg5,
               semg6, semg7, semc0, semc1, semc2, semc3, semc4, semc5,
               semc6, semc7):
    c = lax.axis_index("c")
    s = lax.axis_index("s")
    w0 = 2 * s
    w1 = 2 * s + 1

    pltpu.sync_copy(cnt_ref.at[w0], cbuf)
    cb0 = cbuf[...]
    pltpu.sync_copy(cnt_ref.at[w1], cbuf)
    cb1 = cbuf[...]
    n0 = jnp.where(c == 0, cb0[0], cb0[1])
    n1 = jnp.where(c == 0, cb1[0], cb1[1])
    ntot = n0 + n1

    rows = [rows0, rows1, rows2, rows3, rows4, rows5, rows6, rows7]
    semg = [semg0, semg1, semg2, semg3, semg4, semg5, semg6, semg7]
    semc = [semc0, semc1, semc2, semc3, semc4, semc5, semc6, semc7]

    def idx_start(j):
        jj = lax.rem(j, IDEPTH)

        @pl.when(j < n0)
        def _():
            pltpu.async_copy(bint_ref.at[c, w0, j], sd.at[jj], semi)

        @pl.when(j >= n0)
        def _():
            pltpu.async_copy(bint_ref.at[c, w1, j - n0], sd.at[jj], semi)

    def idx_wait(j):
        jj = lax.rem(j, IDEPTH)
        pltpu.make_async_copy(bint_ref.at[0, 0, 0], sd.at[jj], semi).wait()

    def gstart(j, k):
        jj = lax.rem(j, IDEPTH)
        pltpu.async_copy(g_ref.at[sd.at[jj, 0]], rows[k], semg[k])

    def gwait(k):
        pltpu.make_async_copy(g_ref.at[pl.ds(0, CP)], rows[k], semg[k]).wait()

    def scstart(j, k):
        jj = lax.rem(j, IDEPTH)
        pltpu.async_copy(rows[k], acc.at[sd.at[jj, 1]], semc[k], add=True)

    def scwait(k):
        pltpu.make_async_copy(rows[k], acc.at[sd.at[0, 1]], semc[k]).wait()

    def zrow(r, carry):
        for q in range(F // 16):
            rows0[r, pl.ds(q * 16, 16)] = jnp.zeros((16,), jnp.float32)
        return carry

    lax.fori_loop(0, CP, zrow, 0)
    base_r = s * RPH
    for t in range(RPH // CP):
        pltpu.sync_copy(rows0, acc.at[pl.ds(base_r + t * CP, CP)])
    if RPH % CP:
        pltpu.sync_copy(rows0.at[pl.ds(0, RPH % CP)],
                        acc.at[pl.ds(base_r + (RPH // CP) * CP, RPH % CP)])
    plsc.subcore_barrier()

    for k in range(DEPTH):
        @pl.when(k < ntot)
        def _(k=k):
            idx_start(k)

    jm = (ntot + (DEPTH - 1)) >> DPL

    def body(J, carry):
        b = J * DEPTH
        for k in range(DEPTH):
            @pl.when(b + k - DEPTH >= 0)
            def _(k=k):
                scwait(k)
        for k in range(DEPTH):
            j = b + k

            @pl.when(j < ntot)
            def _(j=j, k=k):
                idx_wait(j)
                gstart(j, k)
        for k in range(DEPTH):
            j = b + k

            @pl.when(j + DEPTH < ntot)
            def _(j=j):
                idx_start(j + DEPTH)
        for k in range(DEPTH):
            j = b + k

            @pl.when(j < ntot)
            def _(j=j, k=k):
                gwait(k)
                scstart(j, k)
        return carry

    lax.fori_loop(0, jm, body, 0)
    base = (jm - 1) * DEPTH
    for k in range(DEPTH):
        @pl.when(jnp.logical_and(base + k >= 0, base + k < ntot))
        def _(k=k):
            scwait(k)

    plsc.subcore_barrier()
    pltpu.sync_copy(acc.at[pl.ds(base_r, RPH)], parts_ref.at[c, pl.ds(base_r, RPH)])


_prop = functools.partial(
    pl.kernel,
    out_type=jax.ShapeDtypeStruct((NC, NPH, F), jnp.float32),
    mesh=_mesh,
    scratch_types=(
        [pltpu.VMEM((IDEPTH, 2, CP), jnp.int32)]
        + [pltpu.VMEM((CP, F), jnp.float32) for _ in range(DEPTH)]
        + [pltpu.VMEM((16,), jnp.int32),
           pltpu.VMEM_SHARED((NPH, F), jnp.float32)]
        + [pltpu.SemaphoreType.DMA for _ in range(2 * DEPTH + 1)]
    ),
)(_prop_body)


_BR = 200
_GRID = N // _BR
_BPH = HALF // _BR

_ROWS_F = pl.BlockSpec((_BR, F), lambda i: (i, 0))
_PARTS = pl.BlockSpec((1, _BR, F), lambda i: (i // _BPH, i % _BPH, 0))
_F_SHAPE = jax.ShapeDtypeStruct((N, F), jnp.float32)


def _tca_body(degp_ref, x_ref, d_ref, g_ref):
    deg = degp_ref[0] + degp_ref[1]
    dis = jnp.where(deg > 0.0, lax.rsqrt(jnp.maximum(deg, 1e-12)), 0.0)
    dmat = jnp.broadcast_to(dis, (_BR, F))
    d_ref[...] = dmat
    g_ref[...] = dmat * x_ref[...]


def _tca(degp, x):
    return pl.pallas_call(
        _tca_body,
        grid=(_GRID,),
        in_specs=[
            pl.BlockSpec((NC, _BR, 1), lambda i: (0, i, 0)),
            _ROWS_F,
        ],
        out_specs=[_ROWS_F, _ROWS_F],
        out_shape=[_F_SHAPE, _F_SHAPE],
    )(degp, x)


def _tcb_body(p_ref, d_ref, tx_ref, g_ref):
    tx = -(d_ref[...] * p_ref[0])
    tx_ref[...] = tx
    g_ref[...] = d_ref[...] * tx


def _tcb(parts, d):
    return pl.pallas_call(
        _tcb_body,
        grid=(_GRID,),
        in_specs=[_PARTS, _ROWS_F],
        out_specs=[_ROWS_F, _ROWS_F],
        out_shape=[_F_SHAPE, _F_SHAPE],
    )(parts, d)


def _tcc1_body(x_ref, t1_ref, p_ref, d_ref, w_ref, b_ref, h_ref, g_ref):
    r = -2.0 * d_ref[...] * p_ref[0]
    acc = jnp.dot(x_ref[...], w_ref[0] - w_ref[2],
                  preferred_element_type=jnp.float32)
    acc += jnp.dot(t1_ref[...], w_ref[1], preferred_element_type=jnp.float32)
    acc += jnp.dot(r, w_ref[2], preferred_element_type=jnp.float32)
    acc += b_ref[...]
    h = jnp.maximum(acc, 0.0)
    h_ref[...] = h
    g_ref[...] = d_ref[...] * h


def _tcc1(x, t1, parts, d, w, b):
    return pl.pallas_call(
        _tcc1_body,
        grid=(_GRID,),
        in_specs=[
            _ROWS_F,
            _ROWS_F,
            _PARTS,
            _ROWS_F,
            pl.BlockSpec((3, F, F), lambda i: (0, 0, 0)),
            pl.BlockSpec((1, F), lambda i: (0, 0)),
        ],
        out_specs=[_ROWS_F, _ROWS_F],
        out_shape=[_F_SHAPE, _F_SHAPE],
    )(x, t1, parts, d, w, b)


def _tcc2_body(h_ref, t1_ref, p_ref, d_ref, w_ref, b_ref, o_ref):
    r = -2.0 * d_ref[...] * p_ref[0]
    acc = jnp.dot(h_ref[...], w_ref[0] - w_ref[2],
                  preferred_element_type=jnp.float32)
    acc += jnp.dot(t1_ref[...], w_ref[1], preferred_element_type=jnp.float32)
    acc += jnp.dot(r, w_ref[2], preferred_element_type=jnp.float32)
    acc += b_ref[...]
    m = jnp.max(acc, axis=1, keepdims=True)
    ex = jnp.exp(acc - m)
    lse = jnp.log(jnp.sum(ex, axis=1, keepdims=True)) + m
    o_ref[...] = acc - lse


def _tcc2(h, t1, parts, d, w, b):
    return pl.pallas_call(
        _tcc2_body,
        grid=(_GRID,),
        in_specs=[
            _ROWS_F,
            _ROWS_F,
            _PARTS,
            _ROWS_F,
            pl.BlockSpec((3, F, C), lambda i: (0, 0, 0)),
            pl.BlockSpec((1, C), lambda i: (0, 0)),
        ],
        out_specs=pl.BlockSpec((_BR, C), lambda i: (i, 0)),
        out_shape=jax.ShapeDtypeStruct((N, C), jnp.float32),
    )(h, t1, parts, d, w, b)


def kernel(x, edge_index, W1, b1, W2, b2):
    ei = jnp.pad(edge_index, ((0, 0), (0, EPAD - E)))
    ei_deg = ei.reshape(2, NW, ND, CD)
    degp = _deg_call(ei_deg)
    bint, cnt = _prep_call(ei.reshape(2, NW, EPT))
    bint = bint.reshape(NC, NW, NBLK, 2, CP)
    d, g0 = _tca(degp.reshape(NC, NP, 1), x)
    p = _prop(g0, bint, cnt)
    tx1, g1 = _tcb(p, d)
    q = _prop(g1, bint, cnt)
    h, gh = _tcc1(x, tx1, q, d, W1, b1.reshape(1, F))
    r1 = _prop(gh, bint, cnt)
    ty1, g3 = _tcb(r1, d)
    z = _prop(g3, bint, cnt)
    return _tcc2(h, ty1, z, d, W2, b2.reshape(1, C))
